# baseline (speedup 1.0000x reference)
import jax
import jax.numpy as jnp
from jax import lax
from jax.experimental import pallas as pl
from jax.experimental.pallas import tpu as pltpu
from jax.experimental.pallas import tpu_sc as plsc

N = 10000
NP = 10240
E = 320000
H = 128
G = 64
NC = 2
NS = 16
CH = 128

EPAD = 327680
NCHUNKS = EPAD // CH
NJT = NCHUNKS // (NC * NS)
SL = NJT // 2
STRIPE = NP // NS
NZCH = NP // CH
NBUF = 2
NBUFH = 8


def _sc_prep_body(z_hbm, dst_hbm, ztab_hbm, zeros_hbm, ones_hbm,
                  x_hbm, degp_hbm,
                  z_v, dslab, ones_v, rows_v, hist_sh, sem_g, sem_s):
    cid = lax.axis_index("c")
    sid = lax.axis_index("s")
    wid = cid * NS + sid
    r0 = sid * STRIPE
    c0 = cid * (NCHUNKS // NC) + sid * NJT

    pltpu.sync_copy(zeros_hbm.at[pl.ds(r0, STRIPE)], hist_sh.at[pl.ds(r0, STRIPE)])
    pltpu.sync_copy(dst_hbm.at[pl.ds(c0, NJT)], dslab)
    pltpu.sync_copy(ones_hbm, ones_v)
    plsc.subcore_barrier()

    def hist_group(g, carry):
        for b in range(NBUFH):
            j = g * NBUFH + b
            pltpu.async_copy(ones_v, hist_sh.at[dslab.at[j]], sem_s, add=True)
        for b in range(NBUFH):
            j = g * NBUFH + b
            pltpu.make_async_copy(ones_v, hist_sh.at[dslab.at[j]], sem_s).wait()
        return carry

    lax.fori_loop(0, NJT // NBUFH, hist_group, 0)

    def gath_body(j, carry):
        c = wid + NC * NS * j

        @pl.when(c < NZCH)
        def _():
            base = c * CH
            pltpu.sync_copy(z_hbm.at[pl.ds(base, CH)], z_v)
            pltpu.async_copy(ztab_hbm.at[z_v], rows_v, sem_g).wait()
            pltpu.sync_copy(rows_v, x_hbm.at[pl.ds(base, CH)])
        return carry

    lax.fori_loop(0, (NZCH + NC * NS - 1) // (NC * NS), gath_body, 0)

    plsc.subcore_barrier()
    pltpu.sync_copy(hist_sh.at[pl.ds(r0, STRIPE)],
                    degp_hbm.at[cid, pl.ds(r0, STRIPE)])


_sc_prep = pl.kernel(
    _sc_prep_body,
    out_type=(jax.ShapeDtypeStruct((NP, H), jnp.float32),
              jax.ShapeDtypeStruct((NC, NP, H), jnp.float32)),
    mesh=plsc.VectorSubcoreMesh(core_axis_name="c", subcore_axis_name="s",
                                num_cores=NC, num_subcores=NS),
    scratch_types=(
        pltpu.VMEM((CH,), jnp.int32),
        pltpu.VMEM((NJT, CH), jnp.int32),
        pltpu.VMEM((CH, H), jnp.float32),
        pltpu.VMEM((CH, H), jnp.float32),
        pltpu.VMEM_SHARED((NP, H), jnp.float32),
        pltpu.SemaphoreType.DMA,
        pltpu.SemaphoreType.DMA,
    ),
)


def _sc_agg_body(yp_hbm, src_hbm, dst_hbm, zeros_hbm,
                 accp_hbm,
                 sslab, dslab, rb0, rb1, acc_sh, sm0, sm1, ss0, ss1):
    cid = lax.axis_index("c")
    sid = lax.axis_index("s")
    r0 = sid * STRIPE
    c0 = cid * (NCHUNKS // NC) + sid * NJT
    rows = (rb0, rb1)
    sems = (sm0, sm1)
    sems_s = (ss0, ss1)

    pltpu.sync_copy(zeros_hbm.at[pl.ds(r0, STRIPE)], acc_sh.at[pl.ds(r0, STRIPE)])
    plsc.subcore_barrier()

    for ph in range(NJT // SL):
        pltpu.sync_copy(src_hbm.at[pl.ds(c0 + ph * SL, SL)], sslab)
        pltpu.sync_copy(dst_hbm.at[pl.ds(c0 + ph * SL, SL)], dslab)

        pltpu.async_copy(yp_hbm.at[sslab.at[0]], rows[0], sems[0])

        def group(g, carry):
            for b in range(NBUF):
                j = g * NBUF + b
                bn = (b + 1) % NBUF
                pltpu.make_async_copy(yp_hbm.at[sslab.at[j]], rows[b],
                                      sems[b]).wait()
                pltpu.async_copy(rows[b], acc_sh.at[dslab.at[j]], sems_s[b],
                                 add=True)

                @pl.when(j + 1 < SL)
                def _():
                    @pl.when(j >= 1)
                    def _():
                        pltpu.make_async_copy(rows[bn],
                                              acc_sh.at[dslab.at[j - 1]],
                                              sems_s[bn]).wait()
                    pltpu.async_copy(yp_hbm.at[sslab.at[j + 1]], rows[bn],
                                     sems[bn])
            return carry

        lax.fori_loop(0, SL // NBUF, group, 0)

        pltpu.make_async_copy(rows[(SL - 2) % NBUF],
                              acc_sh.at[dslab.at[SL - 2]],
                              sems_s[(SL - 2) % NBUF]).wait()
        pltpu.make_async_copy(rows[(SL - 1) % NBUF],
                              acc_sh.at[dslab.at[SL - 1]],
                              sems_s[(SL - 1) % NBUF]).wait()

    plsc.subcore_barrier()
    pltpu.sync_copy(acc_sh.at[pl.ds(r0, STRIPE)],
                    accp_hbm.at[cid, pl.ds(r0, STRIPE)])


_sc_agg = pl.kernel(
    _sc_agg_body,
    out_type=jax.ShapeDtypeStruct((NC, NP, H), jnp.float32),
    mesh=plsc.VectorSubcoreMesh(core_axis_name="c", subcore_axis_name="s",
                                num_cores=NC, num_subcores=NS),
    scratch_types=(
        pltpu.VMEM((SL, CH), jnp.int32),
        pltpu.VMEM((SL, CH), jnp.int32),
        pltpu.VMEM((CH, H), jnp.float32),
        pltpu.VMEM((CH, H), jnp.float32),
        pltpu.VMEM_SHARED((NP, H), jnp.float32),
        pltpu.SemaphoreType.DMA,
        pltpu.SemaphoreType.DMA,
        pltpu.SemaphoreType.DMA,
        pltpu.SemaphoreType.DMA,
    ),
)


BR = 1024


def _tc_first_body(x_ref, degp_ref, w_ref, yp_ref, dinv_ref):
    dp = degp_ref[...]
    deg = 1.0 + dp[0, :, 0:1] + dp[1, :, 0:1]
    dinv = lax.rsqrt(deg)
    y = jnp.dot(x_ref[...], w_ref[...], preferred_element_type=jnp.float32)
    yp_ref[...] = y * dinv
    dinv_ref[...] = dinv


_tc_first = pl.pallas_call(
    _tc_first_body,
    grid=(NP // BR,),
    in_specs=[
        pl.BlockSpec((BR, H), lambda i: (i, 0)),
        pl.BlockSpec((NC, BR, H), lambda i: (0, i, 0)),
        pl.BlockSpec((H, H), lambda i: (0, 0)),
    ],
    out_specs=[
        pl.BlockSpec((BR, H), lambda i: (i, 0)),
        pl.BlockSpec((BR, 1), lambda i: (i, 0)),
    ],
    out_shape=[jax.ShapeDtypeStruct((NP, H), jnp.float32),
               jax.ShapeDtypeStruct((NP, 1), jnp.float32)],
)


def _tc_mid_body(accp_ref, yp_ref, dinv_ref, b_ref, w_ref, out_ref):
    a = accp_ref[...]
    dinv = dinv_ref[...]
    x = (a[0] + a[1] + yp_ref[...]) * dinv + b_ref[...]
    x = jnp.maximum(x, 0.0)
    out_ref[...] = jnp.dot(x, w_ref[...],
                           preferred_element_type=jnp.float32) * dinv


_tc_mid = pl.pallas_call(
    _tc_mid_body,
    grid=(NP // BR,),
    in_specs=[
        pl.BlockSpec((NC, BR, H), lambda i: (0, i, 0)),
        pl.BlockSpec((BR, H), lambda i: (i, 0)),
        pl.BlockSpec((BR, 1), lambda i: (i, 0)),
        pl.BlockSpec((1, H), lambda i: (0, 0)),
        pl.BlockSpec((H, H), lambda i: (0, 0)),
    ],
    out_specs=pl.BlockSpec((BR, H), lambda i: (i, 0)),
    out_shape=jax.ShapeDtypeStruct((NP, H), jnp.float32),
)


def _tc_pool_body(accp_ref, yp_ref, dinv_ref, b2_ref, bat_ref,
                  l1w_ref, l1b_ref, l2w_ref, l2b_ref, out_ref):
    a = accp_ref[...]
    x3 = (a[0] + a[1] + yp_ref[...]) * dinv_ref[...] + b2_ref[...]

    bat = bat_ref[...]
    gids = lax.broadcasted_iota(jnp.int32, (G, 1), 0)
    m = (bat < gids).astype(jnp.float32)
    center = jnp.sum(m, axis=1, keepdims=True).astype(jnp.int32)
    cs = jnp.minimum(center, N - 1)
    cd = jnp.minimum(center + 1, N - 1)
    cols = lax.broadcasted_iota(jnp.int32, (1, NP), 1)
    ohs = (cols == cs).astype(jnp.float32)
    ohd = (cols == cd).astype(jnp.float32)
    xs = jnp.dot(ohs, x3, preferred_element_type=jnp.float32)
    xd = jnp.dot(ohd, x3, preferred_element_type=jnp.float32)
    h = xs * xd
    h = jnp.maximum(
        jnp.dot(h, l1w_ref[...], preferred_element_type=jnp.float32)
        + l1b_ref[...], 0.0)
    out_ref[...] = (jnp.dot(h, l2w_ref[...], preferred_element_type=jnp.float32)
                    + l2b_ref[...])


_tc_pool = pl.pallas_call(
    _tc_pool_body,
    out_shape=jax.ShapeDtypeStruct((G, 1), jnp.float32),
)


def kernel(z, edge_index, batch, z_table, W0, b0, W1, b1, W2, b2,
           lin1_W, lin1_b, lin2_W, lin2_b):
    z_pad = jnp.concatenate(
        [z.astype(jnp.int32), jnp.zeros((NP - N,), jnp.int32)])
    pad_i = jnp.arange(EPAD - E, dtype=jnp.int32)
    src = jnp.concatenate(
        [edge_index[0].astype(jnp.int32), pad_i % N]).reshape(-1, CH)
    dst = jnp.concatenate(
        [edge_index[1].astype(jnp.int32), N + pad_i % (NP - N)]).reshape(-1, CH)
    bat_row = jnp.concatenate(
        [batch.astype(jnp.int32), jnp.full((NP - N,), G + 1, jnp.int32)]
    ).reshape(1, NP)

    zeros_nh = jnp.zeros((NP, H), jnp.float32)
    ones_ch = jnp.ones((CH, H), jnp.float32)

    x, degp = _sc_prep(z_pad, dst, z_table, zeros_nh, ones_ch)
    y0p, dinv = _tc_first(x, degp, W0)
    acc0 = _sc_agg(y0p, src, dst, zeros_nh)
    y1p = _tc_mid(acc0, y0p, dinv, b0.reshape(1, H), W1)
    acc1 = _sc_agg(y1p, src, dst, zeros_nh)
    y2p = _tc_mid(acc1, y1p, dinv, b1.reshape(1, H), W2)
    acc2 = _sc_agg(y2p, src, dst, zeros_nh)
    out = _tc_pool(acc2, y2p, dinv, b2.reshape(1, H), bat_row,
                   lin1_W, lin1_b.reshape(1, H), lin2_W,
                   lin2_b.reshape(1, 1))
    return out


# speedup vs baseline: 1.1313x; 1.1313x over previous
import jax
import jax.numpy as jnp
from jax import lax
from jax.experimental import pallas as pl
from jax.experimental.pallas import tpu as pltpu
from jax.experimental.pallas import tpu_sc as plsc

N = 10000
NP = 10240
E = 320000
H = 128
G = 64
NC = 2
NS = 16
CH = 128

EPAD = 327680
NCHUNKS = EPAD // CH
NJT = NCHUNKS // (NC * NS)
SL = NJT // 2
STRIPE = NP // NS
NZCH = NP // CH
NBUF = 2
NBUFH = 8


def _sc_prep_body(z_hbm, dst_hbm, ztab_hbm, zeros_hbm, ones_hbm,
                  x_hbm, degp_hbm,
                  z_v, dslab, ones_v, rows_v, hist_sh, sem_g, sem_s):
    cid = lax.axis_index("c")
    sid = lax.axis_index("s")
    wid = cid * NS + sid
    r0 = sid * STRIPE
    c0 = cid * (NCHUNKS // NC) + sid * NJT

    pltpu.sync_copy(zeros_hbm.at[pl.ds(r0, STRIPE)], hist_sh.at[pl.ds(r0, STRIPE)])
    pltpu.sync_copy(dst_hbm.at[pl.ds(c0, NJT)], dslab)
    pltpu.sync_copy(ones_hbm, ones_v)
    plsc.subcore_barrier()

    def hist_group(g, carry):
        for b in range(NBUFH):
            j = g * NBUFH + b
            pltpu.async_copy(ones_v, hist_sh.at[dslab.at[j]], sem_s, add=True)
        for b in range(NBUFH):
            j = g * NBUFH + b
            pltpu.make_async_copy(ones_v, hist_sh.at[dslab.at[j]], sem_s).wait()
        return carry

    lax.fori_loop(0, NJT // NBUFH, hist_group, 0)

    def gath_body(j, carry):
        c = wid + NC * NS * j

        @pl.when(c < NZCH)
        def _():
            base = c * CH
            pltpu.sync_copy(z_hbm.at[pl.ds(base, CH)], z_v)
            pltpu.async_copy(ztab_hbm.at[z_v], rows_v, sem_g).wait()
            pltpu.sync_copy(rows_v, x_hbm.at[pl.ds(base, CH)])
        return carry

    lax.fori_loop(0, (NZCH + NC * NS - 1) // (NC * NS), gath_body, 0)

    plsc.subcore_barrier()
    pltpu.sync_copy(hist_sh.at[pl.ds(r0, STRIPE)],
                    degp_hbm.at[cid, pl.ds(r0, STRIPE)])


_sc_prep = pl.kernel(
    _sc_prep_body,
    out_type=(jax.ShapeDtypeStruct((NP, H), jnp.float32),
              jax.ShapeDtypeStruct((NC, NP, H), jnp.float32)),
    mesh=plsc.VectorSubcoreMesh(core_axis_name="c", subcore_axis_name="s",
                                num_cores=NC, num_subcores=NS),
    scratch_types=(
        pltpu.VMEM((CH,), jnp.int32),
        pltpu.VMEM((NJT, CH), jnp.int32),
        pltpu.VMEM((CH, H), jnp.float32),
        pltpu.VMEM((CH, H), jnp.float32),
        pltpu.VMEM_SHARED((NP, H), jnp.float32),
        pltpu.SemaphoreType.DMA,
        pltpu.SemaphoreType.DMA,
    ),
)


def _sc_agg_body(yp_hbm, src_hbm, dst_hbm, zeros_hbm,
                 accp_hbm,
                 sslab, dslab, rb0, rb1, acc_sh, sm0, sm1):
    cid = lax.axis_index("c")
    sid = lax.axis_index("s")
    r0 = sid * STRIPE
    c0 = cid * (NCHUNKS // NC) + sid * NJT
    rows = (rb0, rb1)
    sems = (sm0, sm1)

    pltpu.sync_copy(zeros_hbm.at[pl.ds(r0, STRIPE)], acc_sh.at[pl.ds(r0, STRIPE)])
    plsc.subcore_barrier()

    for ph in range(NJT // SL):
        pltpu.sync_copy(src_hbm.at[pl.ds(c0 + ph * SL, SL)], sslab)
        pltpu.sync_copy(dst_hbm.at[pl.ds(c0 + ph * SL, SL)], dslab)

        for b in range(NBUF):
            pltpu.async_copy(yp_hbm.at[sslab.at[b]], rows[b], sems[b])

        def group(g, carry):
            for b in range(NBUF):
                j = g * NBUF + b
                pltpu.make_async_copy(yp_hbm.at[sslab.at[j]], rows[b],
                                      sems[b]).wait()
                pltpu.sync_copy(rows[b], acc_sh.at[dslab.at[j]], add=True)

                @pl.when(j + NBUF < SL)
                def _():
                    pltpu.async_copy(yp_hbm.at[sslab.at[j + NBUF]], rows[b],
                                     sems[b])
            return carry

        lax.fori_loop(0, SL // NBUF, group, 0)

    plsc.subcore_barrier()
    pltpu.sync_copy(acc_sh.at[pl.ds(r0, STRIPE)],
                    accp_hbm.at[cid, pl.ds(r0, STRIPE)])


_sc_agg = pl.kernel(
    _sc_agg_body,
    out_type=jax.ShapeDtypeStruct((NC, NP, H), jnp.float32),
    mesh=plsc.VectorSubcoreMesh(core_axis_name="c", subcore_axis_name="s",
                                num_cores=NC, num_subcores=NS),
    scratch_types=(
        pltpu.VMEM((SL, CH), jnp.int32),
        pltpu.VMEM((SL, CH), jnp.int32),
        pltpu.VMEM((CH, H), jnp.float32),
        pltpu.VMEM((CH, H), jnp.float32),
        pltpu.VMEM_SHARED((NP, H), jnp.float32),
        pltpu.SemaphoreType.DMA,
        pltpu.SemaphoreType.DMA,
    ),
)


BR = 1024


def _tc_first_body(x_ref, degp_ref, w_ref, yp_ref, dinv_ref):
    dp = degp_ref[...]
    deg = 1.0 + dp[0, :, 0:1] + dp[1, :, 0:1]
    dinv = lax.rsqrt(deg)
    y = jnp.dot(x_ref[...], w_ref[...], preferred_element_type=jnp.float32)
    yp_ref[...] = y * dinv
    dinv_ref[...] = dinv


_tc_first = pl.pallas_call(
    _tc_first_body,
    grid=(NP // BR,),
    in_specs=[
        pl.BlockSpec((BR, H), lambda i: (i, 0)),
        pl.BlockSpec((NC, BR, H), lambda i: (0, i, 0)),
        pl.BlockSpec((H, H), lambda i: (0, 0)),
    ],
    out_specs=[
        pl.BlockSpec((BR, H), lambda i: (i, 0)),
        pl.BlockSpec((BR, 1), lambda i: (i, 0)),
    ],
    out_shape=[jax.ShapeDtypeStruct((NP, H), jnp.float32),
               jax.ShapeDtypeStruct((NP, 1), jnp.float32)],
)


def _tc_mid_body(accp_ref, yp_ref, dinv_ref, b_ref, w_ref, out_ref):
    a = accp_ref[...]
    dinv = dinv_ref[...]
    x = (a[0] + a[1] + yp_ref[...]) * dinv + b_ref[...]
    x = jnp.maximum(x, 0.0)
    out_ref[...] = jnp.dot(x, w_ref[...],
                           preferred_element_type=jnp.float32) * dinv


_tc_mid = pl.pallas_call(
    _tc_mid_body,
    grid=(NP // BR,),
    in_specs=[
        pl.BlockSpec((NC, BR, H), lambda i: (0, i, 0)),
        pl.BlockSpec((BR, H), lambda i: (i, 0)),
        pl.BlockSpec((BR, 1), lambda i: (i, 0)),
        pl.BlockSpec((1, H), lambda i: (0, 0)),
        pl.BlockSpec((H, H), lambda i: (0, 0)),
    ],
    out_specs=pl.BlockSpec((BR, H), lambda i: (i, 0)),
    out_shape=jax.ShapeDtypeStruct((NP, H), jnp.float32),
)


def _tc_pool_body(accp_ref, yp_ref, dinv_ref, b2_ref, bat_ref,
                  l1w_ref, l1b_ref, l2w_ref, l2b_ref, out_ref):
    a = accp_ref[...]
    x3 = (a[0] + a[1] + yp_ref[...]) * dinv_ref[...] + b2_ref[...]

    bat = bat_ref[...]
    gids = lax.broadcasted_iota(jnp.int32, (G, 1), 0)
    m = (bat < gids).astype(jnp.float32)
    center = jnp.sum(m, axis=1, keepdims=True).astype(jnp.int32)
    cs = jnp.minimum(center, N - 1)
    cd = jnp.minimum(center + 1, N - 1)
    cols = lax.broadcasted_iota(jnp.int32, (1, NP), 1)
    ohs = (cols == cs).astype(jnp.float32)
    ohd = (cols == cd).astype(jnp.float32)
    xs = jnp.dot(ohs, x3, preferred_element_type=jnp.float32)
    xd = jnp.dot(ohd, x3, preferred_element_type=jnp.float32)
    h = xs * xd
    h = jnp.maximum(
        jnp.dot(h, l1w_ref[...], preferred_element_type=jnp.float32)
        + l1b_ref[...], 0.0)
    out_ref[...] = (jnp.dot(h, l2w_ref[...], preferred_element_type=jnp.float32)
                    + l2b_ref[...])


_tc_pool = pl.pallas_call(
    _tc_pool_body,
    out_shape=jax.ShapeDtypeStruct((G, 1), jnp.float32),
)


def kernel(z, edge_index, batch, z_table, W0, b0, W1, b1, W2, b2,
           lin1_W, lin1_b, lin2_W, lin2_b):
    z_pad = jnp.concatenate(
        [z.astype(jnp.int32), jnp.zeros((NP - N,), jnp.int32)])
    pad_i = jnp.arange(EPAD - E, dtype=jnp.int32)
    src = jnp.concatenate(
        [edge_index[0].astype(jnp.int32), pad_i % N]).reshape(-1, CH)
    dst = jnp.concatenate(
        [edge_index[1].astype(jnp.int32), N + pad_i % (NP - N)]).reshape(-1, CH)
    bat_row = jnp.concatenate(
        [batch.astype(jnp.int32), jnp.full((NP - N,), G + 1, jnp.int32)]
    ).reshape(1, NP)

    zeros_nh = jnp.zeros((NP, H), jnp.float32)
    ones_ch = jnp.ones((CH, H), jnp.float32)

    x, degp = _sc_prep(z_pad, dst, z_table, zeros_nh, ones_ch)
    y0p, dinv = _tc_first(x, degp, W0)
    acc0 = _sc_agg(y0p, src, dst, zeros_nh)
    y1p = _tc_mid(acc0, y0p, dinv, b0.reshape(1, H), W1)
    acc1 = _sc_agg(y1p, src, dst, zeros_nh)
    y2p = _tc_mid(acc1, y1p, dinv, b1.reshape(1, H), W2)
    acc2 = _sc_agg(y2p, src, dst, zeros_nh)
    out = _tc_pool(acc2, y2p, dinv, b2.reshape(1, H), bat_row,
                   lin1_W, lin1_b.reshape(1, H), lin2_W,
                   lin2_b.reshape(1, 1))
    return out
